# baseline (speedup 1.0000x reference)
import functools

import jax
import jax.numpy as jnp
from jax import lax
from jax.experimental import pallas as pl
from jax.experimental.pallas import tpu as pltpu
from jax.experimental.pallas import tpu_sc as plsc

Z = 64
B = 16384
NC = 2
NS = 16
NW = NC * NS
BPW = B // NW
CH = 128
NCH = BPW // CH


@functools.partial(
    pl.kernel,
    mesh=plsc.VectorSubcoreMesh(core_axis_name="c", subcore_axis_name="s"),
    compiler_params=pltpu.CompilerParams(use_tc_tiling_on_sc=False),
    out_type=(
        jax.ShapeDtypeStruct((NW, NCH, CH, Z), jnp.float32),
        jax.ShapeDtypeStruct((NW, NCH, CH, Z), jnp.float32),
    ),
    scratch_types=[
        pltpu.VMEM((NCH, CH), jnp.int32),
        pltpu.VMEM((NCH, CH, Z), jnp.float32),
        pltpu.VMEM((NCH, CH, Z), jnp.float32),
        pltpu.SemaphoreType.DMA,
        pltpu.SemaphoreType.DMA,
    ],
)
def _gather2(u_hbm, mw_hbm, vw_hbm, om_hbm, ov_hbm,
             idx_v, mrows, vrows, sem_m, sem_v):
    wid = lax.axis_index("s") * NC + lax.axis_index("c")
    pltpu.sync_copy(u_hbm.at[wid], idx_v)
    copies = []
    for j in range(NCH):
        copies.append(pltpu.async_copy(mw_hbm.at[idx_v.at[j]], mrows.at[j], sem_m))
        copies.append(pltpu.async_copy(vw_hbm.at[idx_v.at[j]], vrows.at[j], sem_v))
    for c in copies:
        c.wait()
    pltpu.sync_copy(mrows, om_hbm.at[wid])
    pltpu.sync_copy(vrows, ov_hbm.at[wid])


def kernel(u_input, lam_mean_w, lam_log_var_w):
    u = u_input.astype(jnp.int32).reshape(NW, NCH, CH)
    om, ov = _gather2(u, lam_mean_w, lam_log_var_w)
    return om.reshape(B, Z), ov.reshape(B, Z)


# speedup vs baseline: 1.5824x; 1.5824x over previous
import functools

import jax
import jax.numpy as jnp
from jax import lax
from jax.experimental import pallas as pl
from jax.experimental.pallas import tpu as pltpu
from jax.experimental.pallas import tpu_sc as plsc

Z = 64
B = 16384
NC = 2
NS = 16
NW = NC * NS
BPW = B // NW
NH = 2
HW = BPW // NH


@functools.partial(
    pl.kernel,
    mesh=plsc.VectorSubcoreMesh(core_axis_name="c", subcore_axis_name="s"),
    out_type=(
        jax.ShapeDtypeStruct((NW, NH, HW, Z), jnp.float32),
        jax.ShapeDtypeStruct((NW, NH, HW, Z), jnp.float32),
    ),
    scratch_types=[
        pltpu.VMEM((BPW,), jnp.int32),
        pltpu.VMEM((HW, Z), jnp.float32),
        pltpu.VMEM((HW, Z), jnp.float32),
        pltpu.SemaphoreType.DMA,
        pltpu.SemaphoreType.DMA,
    ],
)
def _gather2(u_hbm, mw_hbm, vw_hbm, om_hbm, ov_hbm,
             idx_v, mrows, vrows, sem_m, sem_v):
    wid = lax.axis_index("s") * NC + lax.axis_index("c")
    pltpu.sync_copy(u_hbm.at[wid], idx_v)

    for h in range(NH):
        def issue(g, _):
            base = g * 16
            v = idx_v[pl.ds(h * HW + base, 16)]
            for l in range(16):
                r = v[l]
                pltpu.async_copy(mw_hbm.at[r], mrows.at[base + l], sem_m)
                pltpu.async_copy(vw_hbm.at[r], vrows.at[base + l], sem_v)
            return 0

        lax.fori_loop(0, HW // 16, issue, 0)
        pltpu.make_async_copy(mw_hbm.at[pl.ds(0, HW)], mrows, sem_m).wait()
        pltpu.make_async_copy(vw_hbm.at[pl.ds(0, HW)], vrows, sem_v).wait()
        pltpu.sync_copy(mrows, om_hbm.at[wid, h])
        pltpu.sync_copy(vrows, ov_hbm.at[wid, h])


def kernel(u_input, lam_mean_w, lam_log_var_w):
    u = u_input.astype(jnp.int32).reshape(NW, BPW)
    om, ov = _gather2(u, lam_mean_w, lam_log_var_w)
    return om.reshape(B, Z), ov.reshape(B, Z)


# speedup vs baseline: 1.7717x; 1.1196x over previous
import functools

import jax
import jax.numpy as jnp
from jax import lax
from jax.experimental import pallas as pl
from jax.experimental.pallas import tpu as pltpu
from jax.experimental.pallas import tpu_sc as plsc

Z = 64
B = 16384
NU = 1_000_000
NC = 2
NS = 16
NW = NC * NS
BPW = B // NW
LB = 128


@functools.partial(
    pl.kernel,
    mesh=plsc.VectorSubcoreMesh(core_axis_name="c", subcore_axis_name="s"),
    compiler_params=pltpu.CompilerParams(needs_layout_passes=False),
    out_type=(
        jax.ShapeDtypeStruct((Z, B), jnp.float32),
        jax.ShapeDtypeStruct((Z, B), jnp.float32),
    ),
    scratch_types=[
        pltpu.VMEM((BPW,), jnp.int32),
        pltpu.VMEM((2, Z, LB), jnp.float32),
        pltpu.VMEM((2, Z, LB), jnp.float32),
        pltpu.VMEM((Z, BPW), jnp.float32),
        pltpu.VMEM((Z, BPW), jnp.float32),
        pltpu.SemaphoreType.DMA,
        pltpu.SemaphoreType.DMA,
        pltpu.SemaphoreType.DMA,
        pltpu.SemaphoreType.DMA,
    ],
)
def _gather2(u_hbm, mwt_hbm, vwt_hbm, omt_hbm, ovt_hbm,
             idx_v, mblk, vblk, mcols, vcols, s0, s1, s2, s3):
    wid = lax.axis_index("s") * NC + lax.axis_index("c")
    base = wid * BPW
    pltpu.sync_copy(u_hbm.at[wid], idx_v)
    msems = (s0, s1)
    vsems = (s2, s3)
    lanes = lax.iota(jnp.int32, 16)

    def issue_one(r, slot):
        al = pl.multiple_of((r >> 7) << 7, LB)
        pltpu.async_copy(mwt_hbm.at[:, pl.ds(al, LB)], mblk.at[slot], msems[slot])
        pltpu.async_copy(vwt_hbm.at[:, pl.ds(al, LB)], vblk.at[slot], vsems[slot])

    def wait_one(slot):
        pltpu.make_async_copy(mwt_hbm.at[:, pl.ds(0, LB)], mblk.at[slot],
                              msems[slot]).wait()
        pltpu.make_async_copy(vwt_hbm.at[:, pl.ds(0, LB)], vblk.at[slot],
                              vsems[slot]).wait()

    def extract_one(r, slot, pos):
        ln = jnp.full((16,), r & (LB - 1), jnp.int32)
        sl = jnp.full((16,), slot, jnp.int32)
        po = jnp.full((16,), pos, jnp.int32)
        for g in range(Z // 16):
            rows = lanes + g * 16
            mvals = plsc.load_gather(mblk, [sl, rows, ln])
            vvals = plsc.load_gather(vblk, [sl, rows, ln])
            plsc.store_scatter(mcols, [rows, po], mvals)
            plsc.store_scatter(vcols, [rows, po], vvals)

    def grp(g, _):
        v = idx_v[pl.ds(g * 16, 16)]
        vn = idx_v[pl.ds(jnp.minimum((g + 1) * 16, BPW - 16), 16)]
        for l in range(16):
            slot = l % 2
            wait_one(slot)
            issue_one(v[l + 1] if l < 15 else vn[0], 1 - slot)
            extract_one(v[l], slot, g * 16 + l)
        return 0

    v0 = idx_v[pl.ds(0, 16)]
    issue_one(v0[0], 0)
    lax.fori_loop(0, BPW // 16, grp, 0)
    wait_one(0)

    pltpu.sync_copy(mcols, omt_hbm.at[:, pl.ds(base, BPW)])
    pltpu.sync_copy(vcols, ovt_hbm.at[:, pl.ds(base, BPW)])


def kernel(u_input, lam_mean_w, lam_log_var_w):
    u = u_input.astype(jnp.int32).reshape(NW, BPW)
    omt, ovt = _gather2(u, lam_mean_w.T, lam_log_var_w.T)
    return omt.T, ovt.T


# speedup vs baseline: 2.6901x; 1.5184x over previous
import functools

import jax
import jax.numpy as jnp
from jax import lax
from jax.experimental import pallas as pl
from jax.experimental.pallas import tpu as pltpu
from jax.experimental.pallas import tpu_sc as plsc

Z = 64
B = 16384
NU = 1_000_000
NC = 2
NS = 16
NW = NC * NS
BPW = B // NW
LB = 128
NSLOT = 4
LA = NSLOT - 1
GPC = 8
NCHK = BPW // LB


@functools.partial(
    pl.kernel,
    mesh=plsc.VectorSubcoreMesh(core_axis_name="c", subcore_axis_name="s"),
    compiler_params=pltpu.CompilerParams(needs_layout_passes=False),
    out_type=(
        jax.ShapeDtypeStruct((Z, B), jnp.float32),
        jax.ShapeDtypeStruct((Z, B), jnp.float32),
    ),
    scratch_types=[
        pltpu.VMEM((BPW,), jnp.int32),
        pltpu.VMEM((NSLOT, Z, LB), jnp.float32),
        pltpu.VMEM((NSLOT, Z, LB), jnp.float32),
        pltpu.VMEM((2, Z, LB), jnp.float32),
        pltpu.VMEM((2, Z, LB), jnp.float32),
    ] + [pltpu.SemaphoreType.DMA] * (2 * NSLOT + 2),
)
def _gather2(u_hbm, mwt_hbm, vwt_hbm, omt_hbm, ovt_hbm,
             idx_v, mblk, vblk, mob, vob, *sems):
    wid = lax.axis_index("s") * NC + lax.axis_index("c")
    base = wid * BPW
    pltpu.sync_copy(u_hbm.at[wid], idx_v)
    msems = sems[:NSLOT]
    vsems = sems[NSLOT:2 * NSLOT]
    sem_om, sem_ov = sems[2 * NSLOT], sems[2 * NSLOT + 1]
    lanes = lax.iota(jnp.int32, 16)

    def issue_one(r, slot):
        al = pl.multiple_of((r >> 7) << 7, LB)
        pltpu.async_copy(mwt_hbm.at[:, pl.ds(al, LB)], mblk.at[slot], msems[slot])
        pltpu.async_copy(vwt_hbm.at[:, pl.ds(al, LB)], vblk.at[slot], vsems[slot])

    def wait_one(slot):
        pltpu.make_async_copy(mwt_hbm.at[:, pl.ds(0, LB)], mblk.at[slot],
                              msems[slot]).wait()
        pltpu.make_async_copy(vwt_hbm.at[:, pl.ds(0, LB)], vblk.at[slot],
                              vsems[slot]).wait()

    def wait_out():
        pltpu.make_async_copy(mob.at[0], omt_hbm.at[:, pl.ds(base, LB)],
                              sem_om).wait()
        pltpu.make_async_copy(vob.at[0], ovt_hbm.at[:, pl.ds(base, LB)],
                              sem_ov).wait()

    def extract_one(r, slot, cb, pos):
        ln = jnp.full((16,), r & (LB - 1), jnp.int32)
        sl = jnp.full((16,), slot, jnp.int32)
        cbv = jnp.full((16,), cb, jnp.int32)
        po = jnp.full((16,), pos, jnp.int32)
        for gz in range(Z // 16):
            rows = lanes + gz * 16
            mvals = plsc.load_gather(mblk, [sl, rows, ln])
            vvals = plsc.load_gather(vblk, [sl, rows, ln])
            plsc.store_scatter(mob, [cbv, rows, po], mvals)
            plsc.store_scatter(vob, [cbv, rows, po], vvals)

    def chunk(c, _):
        cb = c % 2
        @pl.when(c >= 2)
        def _():
            wait_out()

        def grp(gg, _):
            g = c * GPC + gg
            v = idx_v[pl.ds(g * 16, 16)]
            vn = idx_v[pl.ds(jnp.minimum((g + 1) * 16, BPW - 16), 16)]
            for l in range(16):
                slot = l % NSLOT
                wait_one(slot)
                nxt = v[l + LA] if l + LA < 16 else vn[l + LA - 16]
                issue_one(nxt, (l + LA) % NSLOT)
                extract_one(v[l], slot, cb, gg * 16 + l)
            return 0

        lax.fori_loop(0, GPC, grp, 0)
        off = base + c * LB
        pltpu.async_copy(mob.at[cb], omt_hbm.at[:, pl.ds(off, LB)], sem_om)
        pltpu.async_copy(vob.at[cb], ovt_hbm.at[:, pl.ds(off, LB)], sem_ov)
        return 0

    v0 = idx_v[pl.ds(0, 16)]
    for i in range(LA):
        issue_one(v0[i], i % NSLOT)
    lax.fori_loop(0, NCHK, chunk, 0)
    for i in range(BPW, BPW + LA):
        wait_one(i % NSLOT)
    wait_out()
    wait_out()


def kernel(u_input, lam_mean_w, lam_log_var_w):
    u = u_input.astype(jnp.int32).reshape(NW, BPW)
    omt, ovt = _gather2(u, lam_mean_w.T, lam_log_var_w.T)
    return omt.T, ovt.T


# speedup vs baseline: 2.6989x; 1.0033x over previous
import functools

import jax
import jax.numpy as jnp
from jax import lax
from jax.experimental import pallas as pl
from jax.experimental.pallas import tpu as pltpu
from jax.experimental.pallas import tpu_sc as plsc

Z = 64
B = 16384
NU = 1_000_000
NC = 2
NS = 16
NW = NC * NS
BPW = B // NW
LB = 128
NSLOT = 4
LA = NSLOT - 1
GPC = 8
NCHK = BPW // LB


@functools.partial(
    pl.kernel,
    mesh=plsc.VectorSubcoreMesh(core_axis_name="c", subcore_axis_name="s"),
    compiler_params=pltpu.CompilerParams(needs_layout_passes=False),
    out_type=(
        jax.ShapeDtypeStruct((Z, B), jnp.float32),
        jax.ShapeDtypeStruct((Z, B), jnp.float32),
    ),
    scratch_types=[
        pltpu.VMEM((BPW,), jnp.int32),
        pltpu.VMEM((NSLOT, Z, LB), jnp.float32),
        pltpu.VMEM((NSLOT, Z, LB), jnp.float32),
        pltpu.VMEM((2, Z, LB), jnp.float32),
        pltpu.VMEM((2, Z, LB), jnp.float32),
    ] + [pltpu.SemaphoreType.DMA] * (2 * NSLOT + 2),
)
def _gather2(u_hbm, mwt_hbm, vwt_hbm, omt_hbm, ovt_hbm,
             idx_v, mblk, vblk, mob, vob, *sems):
    wid = lax.axis_index("s") * NC + lax.axis_index("c")
    base = wid * BPW
    pltpu.sync_copy(u_hbm.at[wid], idx_v)
    msems = sems[:NSLOT]
    vsems = sems[NSLOT:2 * NSLOT]
    sem_om, sem_ov = sems[2 * NSLOT], sems[2 * NSLOT + 1]
    lanes = lax.iota(jnp.int32, 16)

    def issue_one(r, slot):
        al = pl.multiple_of((r >> 7) << 7, LB)
        H = Z // 2
        for h in (0, H):
            pltpu.async_copy(mwt_hbm.at[pl.ds(h, H), pl.ds(al, LB)],
                             mblk.at[slot, pl.ds(h, H)], msems[slot])
            pltpu.async_copy(vwt_hbm.at[pl.ds(h, H), pl.ds(al, LB)],
                             vblk.at[slot, pl.ds(h, H)], vsems[slot])

    def wait_one(slot):
        pltpu.make_async_copy(mwt_hbm.at[:, pl.ds(0, LB)], mblk.at[slot],
                              msems[slot]).wait()
        pltpu.make_async_copy(vwt_hbm.at[:, pl.ds(0, LB)], vblk.at[slot],
                              vsems[slot]).wait()

    def wait_out():
        pltpu.make_async_copy(mob.at[0], omt_hbm.at[:, pl.ds(base, LB)],
                              sem_om).wait()
        pltpu.make_async_copy(vob.at[0], ovt_hbm.at[:, pl.ds(base, LB)],
                              sem_ov).wait()

    def extract_one(r, slot, cb, pos):
        ln = jnp.full((16,), r & (LB - 1), jnp.int32)
        sl = jnp.full((16,), slot, jnp.int32)
        cbv = jnp.full((16,), cb, jnp.int32)
        po = jnp.full((16,), pos, jnp.int32)
        for gz in range(Z // 16):
            rows = lanes + gz * 16
            mvals = plsc.load_gather(mblk, [sl, rows, ln])
            vvals = plsc.load_gather(vblk, [sl, rows, ln])
            plsc.store_scatter(mob, [cbv, rows, po], mvals)
            plsc.store_scatter(vob, [cbv, rows, po], vvals)

    def chunk(c, _):
        cb = c % 2
        @pl.when(c >= 2)
        def _():
            wait_out()

        def grp(gg, _):
            g = c * GPC + gg
            v = idx_v[pl.ds(g * 16, 16)]
            vn = idx_v[pl.ds(jnp.minimum((g + 1) * 16, BPW - 16), 16)]
            for l in range(16):
                slot = l % NSLOT
                wait_one(slot)
                nxt = v[l + LA] if l + LA < 16 else vn[l + LA - 16]
                issue_one(nxt, (l + LA) % NSLOT)
                extract_one(v[l], slot, cb, gg * 16 + l)
            return 0

        lax.fori_loop(0, GPC, grp, 0)
        off = base + c * LB
        pltpu.async_copy(mob.at[cb], omt_hbm.at[:, pl.ds(off, LB)], sem_om)
        pltpu.async_copy(vob.at[cb], ovt_hbm.at[:, pl.ds(off, LB)], sem_ov)
        return 0

    v0 = idx_v[pl.ds(0, 16)]
    for i in range(LA):
        issue_one(v0[i], i % NSLOT)
    lax.fori_loop(0, NCHK, chunk, 0)
    for i in range(BPW, BPW + LA):
        wait_one(i % NSLOT)
    wait_out()
    wait_out()


def kernel(u_input, lam_mean_w, lam_log_var_w):
    u = u_input.astype(jnp.int32).reshape(NW, BPW)
    omt, ovt = _gather2(u, lam_mean_w.T, lam_log_var_w.T)
    return omt.T, ovt.T


# speedup vs baseline: 2.6994x; 1.0002x over previous
import functools

import jax
import jax.numpy as jnp
from jax import lax
from jax.experimental import pallas as pl
from jax.experimental.pallas import tpu as pltpu
from jax.experimental.pallas import tpu_sc as plsc

Z = 64
B = 16384
NU = 1_000_000
NC = 2
NS = 16
NW = NC * NS
BPW = B // NW
LB = 128
NSLOT = 4
LA = NSLOT - 1
GPC = 8
NCHK = BPW // LB


@functools.partial(
    pl.kernel,
    mesh=plsc.VectorSubcoreMesh(core_axis_name="c", subcore_axis_name="s"),
    compiler_params=pltpu.CompilerParams(needs_layout_passes=False),
    out_type=(
        jax.ShapeDtypeStruct((Z, B), jnp.float32),
        jax.ShapeDtypeStruct((Z, B), jnp.float32),
    ),
    scratch_types=[
        pltpu.VMEM((BPW,), jnp.int32),
        pltpu.VMEM((NSLOT, Z, LB), jnp.float32),
        pltpu.VMEM((NSLOT, Z, LB), jnp.float32),
        pltpu.VMEM((2, Z, LB), jnp.float32),
        pltpu.VMEM((2, Z, LB), jnp.float32),
    ] + [pltpu.SemaphoreType.DMA] * (2 * NSLOT + 2),
)
def _gather2(u_hbm, mwt_hbm, vwt_hbm, omt_hbm, ovt_hbm,
             idx_v, mblk, vblk, mob, vob, *sems):
    wid = lax.axis_index("s") * NC + lax.axis_index("c")
    base = wid * BPW
    pltpu.sync_copy(u_hbm.at[wid], idx_v)
    msems = sems[:NSLOT]
    vsems = sems[NSLOT:2 * NSLOT]
    sem_om, sem_ov = sems[2 * NSLOT], sems[2 * NSLOT + 1]
    lanes = lax.iota(jnp.int32, 16)

    def issue_one(r, slot):
        al = pl.multiple_of((r >> 7) << 7, LB)
        pltpu.async_copy(mwt_hbm.at[:, pl.ds(al, LB)], mblk.at[slot], msems[slot])
        pltpu.async_copy(vwt_hbm.at[:, pl.ds(al, LB)], vblk.at[slot], vsems[slot])

    def wait_one(slot):
        pltpu.make_async_copy(mwt_hbm.at[:, pl.ds(0, LB)], mblk.at[slot],
                              msems[slot]).wait()
        pltpu.make_async_copy(vwt_hbm.at[:, pl.ds(0, LB)], vblk.at[slot],
                              vsems[slot]).wait()

    def wait_out():
        pltpu.make_async_copy(mob.at[0], omt_hbm.at[:, pl.ds(base, LB)],
                              sem_om).wait()
        pltpu.make_async_copy(vob.at[0], ovt_hbm.at[:, pl.ds(base, LB)],
                              sem_ov).wait()

    def extract_one(r, slot, cb, pos):
        ln = jnp.full((16,), r & (LB - 1), jnp.int32)
        sl = jnp.full((16,), slot, jnp.int32)
        cbv = jnp.full((16,), cb, jnp.int32)
        po = jnp.full((16,), pos, jnp.int32)
        for gz in range(Z // 16):
            rows = lanes + gz * 16
            mvals = plsc.load_gather(mblk, [sl, rows, ln])
            vvals = plsc.load_gather(vblk, [sl, rows, ln])
            plsc.store_scatter(mob, [cbv, rows, po], mvals)
            plsc.store_scatter(vob, [cbv, rows, po], vvals)

    def chunk(c, _):
        cb = c % 2
        @pl.when(c >= 2)
        def _():
            wait_out()

        def grp(gg, _):
            g = c * GPC + gg
            v = idx_v[pl.ds(g * 16, 16)]
            vn = idx_v[pl.ds(jnp.minimum((g + 1) * 16, BPW - 16), 16)]
            for l in range(16):
                slot = l % NSLOT
                wait_one(slot)
                nxt = v[l + LA] if l + LA < 16 else vn[l + LA - 16]
                issue_one(nxt, (l + LA) % NSLOT)
                extract_one(v[l], slot, cb, gg * 16 + l)
            return 0

        lax.fori_loop(0, GPC, grp, 0)
        off = base + c * LB
        pltpu.async_copy(mob.at[cb], omt_hbm.at[:, pl.ds(off, LB)], sem_om)
        pltpu.async_copy(vob.at[cb], ovt_hbm.at[:, pl.ds(off, LB)], sem_ov)
        return 0

    v0 = idx_v[pl.ds(0, 16)]
    for i in range(LA):
        issue_one(v0[i], i % NSLOT)
    lax.fori_loop(0, NCHK, chunk, 0)
    for i in range(BPW, BPW + LA):
        wait_one(i % NSLOT)
    wait_out()
    wait_out()


def kernel(u_input, lam_mean_w, lam_log_var_w):
    u = u_input.astype(jnp.int32).reshape(NW, BPW)
    omt, ovt = _gather2(u, lam_mean_w.T, lam_log_var_w.T)
    return omt.T, ovt.T
